# baseline (speedup 1.0000x reference)
import functools

import jax
import jax.numpy as jnp
from jax import lax
from jax.experimental import pallas as pl
from jax.experimental.pallas import tpu as pltpu
from jax.experimental.pallas import tpu_sc as plsc

N, C, O = 10000, 128, 128
D = 2000
M = 20000
P = 160000

NPAD = 10112
HALF_M = M // 2
HALF_D = D // 2
F32 = jnp.float32
I32 = jnp.int32

_N_CHUNKS = NPAD // 128
P2 = 163840
_P2_CHUNKS = P2 // 128


def _mesh():
  return plsc.VectorSubcoreMesh(core_axis_name="c", subcore_axis_name="s")


def _mask_local(vals_ref, idx_ref, base, size, n):
  for j in range(n // 16):
    sl = pl.ds(j * 16, 16)
    v = vals_ref[sl]
    loc = v - base
    ok = (loc >= 0) & (loc < size)
    idx_ref[sl] = jnp.where(ok, loc, -1)


_MROWS = 20096
_CQ = C // 4


def _phase_a_body(xq0, xq1, xq2, xq3, xpad, srcp, iip, domp, z128, zq, z16,
                  one16,
                  msgq0, msgq1, msgq2, msgq3,
                  xdom0_out, xdom1_out, cnt0_out, cnt1_out,
                  rows_v, rowq0, rowq1, zrow_v, zq_v, z16_v, one16_v,
                  srcv0, srcv1, iiv0, iiv1, gsem0, gsem1, ssem0, ssem1,
                  acc_msg, acc_dom, acc_cnt):
  c = lax.axis_index("c")
  s = lax.axis_index("s")

  pltpu.sync_copy(z128, zrow_v)
  pltpu.sync_copy(zq, zq_v)
  pltpu.sync_copy(z16, z16_v)
  pltpu.sync_copy(one16, one16_v)

  for rnd, (xq, msgq) in enumerate(
      ((xq0, msgq0), (xq1, msgq1), (xq2, msgq2), (xq3, msgq3))):
    def _zmsg(k, carry):
      ch = s + 16 * k
      @pl.when(ch < _MROWS // 128)
      def _():
        pltpu.sync_copy(zq_v, acc_msg.at[pl.ds(128 * ch, 128)])
      return carry
    lax.fori_loop(0, 10, _zmsg, 0)
    plsc.subcore_barrier()

    base_ch = 640 * c + 40 * s
    bufs = ((rowq0, srcv0, iiv0, gsem0, ssem0),
            (rowq1, srcv1, iiv1, gsem1, ssem1))

    def _stage(b, ch):
      rowq, srcv, iiv, gsem, _ = b
      p0 = 128 * ch
      pltpu.sync_copy(srcp.at[pl.ds(p0, 128)], srcv)
      pltpu.sync_copy(iip.at[pl.ds(p0, 128)], iiv)
      pltpu.async_copy(xq.at[srcv], rowq, gsem)

    def _wait_g(b):
      rowq, srcv, _, gsem, _ = b
      pltpu.make_async_copy(xq.at[srcv], rowq, gsem).wait()

    def _scat(b):
      rowq, _, iiv, _, ssem = b
      pltpu.async_copy(rowq, acc_msg.at[iiv], ssem, add=True)

    def _wait_s(b):
      rowq, _, iiv, _, ssem = b
      pltpu.make_async_copy(rowq, acc_msg.at[iiv], ssem).wait()

    def _chunk(k, carry):
      b = bufs[0]
      _stage(b, base_ch + k)
      _wait_g(b)
      rowq, _, iiv, _, _ = b
      pltpu.sync_copy(rowq, acc_msg.at[iiv], add=True)
      return carry
    lax.fori_loop(0, 40, _chunk, 0)
    plsc.subcore_barrier()

    def _wmsg(k, carry):
      ch = s + 16 * k
      @pl.when(ch < M // 128)
      def _():
        pltpu.sync_copy(acc_msg.at[pl.ds(128 * ch, 128)],
                        msgq.at[c, pl.ds(128 * ch, 128)])
      return carry
    lax.fori_loop(0, 10, _wmsg, 0)

    @pl.when(s == 15)
    def _():
      t0 = (M // 128) * 128
      pltpu.sync_copy(acc_msg.at[pl.ds(t0, M - t0)],
                      msgq.at[c, pl.ds(t0, M - t0)])
    plsc.subcore_barrier()

  pltpu.sync_copy(zrow_v, acc_dom.at[pl.ds(128 * s, 128)])

  @pl.when(s == 8)
  def _():
    def _zc(k, carry):
      pltpu.sync_copy(z16_v, acc_cnt.at[pl.ds(128 * k, 128)])
      return carry
    lax.fori_loop(0, 16, _zc, 0)

  plsc.subcore_barrier()

  def _row(k, carry):
    ch = 40 * c + s + 16 * k
    @pl.when((ch < 40 * c + 40) & (ch < _N_CHUNKS))
    def _():
      r0 = 128 * ch
      pltpu.sync_copy(xpad.at[pl.ds(r0, 128)], rows_v)
      pltpu.sync_copy(domp.at[pl.ds(r0, 128)], srcv0)
      pltpu.sync_copy(rows_v, acc_dom.at[srcv0], add=True)
      pltpu.sync_copy(one16_v, acc_cnt.at[srcv0], add=True)
    return carry
  lax.fori_loop(0, 3, _row, 0)

  plsc.subcore_barrier()

  def _wdom(xdom_out, cnt_out):
    @pl.when(s < 15)
    def _():
      pltpu.sync_copy(acc_dom.at[pl.ds(128 * s, 128)],
                      xdom_out.at[pl.ds(128 * s, 128)])
    @pl.when(s == 15)
    def _():
      pltpu.sync_copy(acc_dom.at[pl.ds(1920, D - 1920)],
                      xdom_out.at[pl.ds(1920, D - 1920)])
      pltpu.sync_copy(acc_cnt.at[pl.ds(0, D)], cnt_out.at[pl.ds(0, D)])

  @pl.when(c == 0)
  def _():
    _wdom(xdom0_out, cnt0_out)

  @pl.when(c == 1)
  def _():
    _wdom(xdom1_out, cnt1_out)


def _phase_a(xq, xpad, srcp, iip, domp, z128, zq, z16, one16):
  return pl.kernel(
      _phase_a_body,
      out_type=(
          jax.ShapeDtypeStruct((2, M, _CQ), F32),
          jax.ShapeDtypeStruct((2, M, _CQ), F32),
          jax.ShapeDtypeStruct((2, M, _CQ), F32),
          jax.ShapeDtypeStruct((2, M, _CQ), F32),
          jax.ShapeDtypeStruct((D, C), F32),
          jax.ShapeDtypeStruct((D, C), F32),
          jax.ShapeDtypeStruct((D, 16), F32),
          jax.ShapeDtypeStruct((D, 16), F32),
      ),
      mesh=_mesh(),
      scratch_types=[
          pltpu.VMEM((128, C), F32),
          pltpu.VMEM((128, _CQ), F32),
          pltpu.VMEM((128, _CQ), F32),
          pltpu.VMEM((128, C), F32),
          pltpu.VMEM((128, _CQ), F32),
          pltpu.VMEM((128, 16), F32),
          pltpu.VMEM((128, 16), F32),
          pltpu.VMEM((128,), I32),
          pltpu.VMEM((128,), I32),
          pltpu.VMEM((128,), I32),
          pltpu.VMEM((128,), I32),
          pltpu.SemaphoreType.DMA,
          pltpu.SemaphoreType.DMA,
          pltpu.SemaphoreType.DMA,
          pltpu.SemaphoreType.DMA,
          pltpu.VMEM_SHARED((_MROWS, _CQ), F32),
          pltpu.VMEM_SHARED((2048, C), F32),
          pltpu.VMEM_SHARED((2048, 16), F32),
      ],
      compiler_params=pltpu.CompilerParams(use_tc_tiling_on_sc=False),
      name="at11_phase_a",
  )(xq[0], xq[1], xq[2], xq[3], xpad, srcp, iip, domp, z128, zq, z16, one16)


def _mm2_body(x_ref, wa_ref, wb_ref, oa_ref, ob_ref):
  xb = x_ref[...]
  oa_ref[...] = jnp.dot(xb, wa_ref[...], preferred_element_type=F32)
  ob_ref[...] = jnp.dot(xb, wb_ref[...], preferred_element_type=F32)


def _phase_b1(x, w_id, w1):
  return pl.pallas_call(
      _mm2_body,
      grid=(N // 400,),
      in_specs=[
          pl.BlockSpec((400, C), lambda i: (i, 0)),
          pl.BlockSpec((C, O), lambda i: (0, 0)),
          pl.BlockSpec((C, O), lambda i: (0, 0)),
      ],
      out_specs=[
          pl.BlockSpec((400, O), lambda i: (i, 0)),
          pl.BlockSpec((400, O), lambda i: (i, 0)),
      ],
      out_shape=[
          jax.ShapeDtypeStruct((N, O), F32),
          jax.ShapeDtypeStruct((N, O), F32),
      ],
      name="at11_phase_b1",
  )(x, w_id, w1)


def _b2_body(q0_ref, q1_ref, q2_ref, q3_ref, w_ref, o_ref):
  parts = []
  for q_ref in (q0_ref, q1_ref, q2_ref, q3_ref):
    q = q_ref[...]
    parts.append(q[0] + q[1])
  msg = jnp.concatenate(parts, axis=1)
  o_ref[...] = jnp.dot(msg, w_ref[...], preferred_element_type=F32)


def _phase_b2(msgq, w_cat):
  qspec = pl.BlockSpec((2, 400, _CQ), lambda i: (0, i, 0))
  return pl.pallas_call(
      _b2_body,
      grid=(M // 400,),
      in_specs=[qspec, qspec, qspec, qspec,
                pl.BlockSpec((C, 2 * O), lambda i: (0, 0))],
      out_specs=pl.BlockSpec((400, 2 * O), lambda i: (i, 0)),
      out_shape=jax.ShapeDtypeStruct((M, 2 * O), F32),
      name="at11_phase_b2",
  )(msgq[0], msgq[1], msgq[2], msgq[3], w_cat)


def _b3_body(sum0_ref, sum1_ref, cnt0_ref, cnt1_ref, w3_ref, wv2_ref, wlin_ref,
             q_ref, r_ref, inv_ref):
  cnt = cnt0_ref[...][:, 0:1] + cnt1_ref[...][:, 0:1]
  xd = (sum0_ref[...] + sum1_ref[...]) / jnp.maximum(cnt, 1.0)
  q_ref[...] = jnp.dot(xd, w3_ref[...], preferred_element_type=F32)
  r_ref[...] = jnp.dot(xd, wv2_ref[...], preferred_element_type=F32)
  inv_ref[...] = jnp.dot(xd, wlin_ref[...], preferred_element_type=F32)


def _phase_b3(xdom0, xdom1, cnt0, cnt1, w3, wv2, wlin):
  return pl.pallas_call(
      _b3_body,
      out_shape=[jax.ShapeDtypeStruct((D, O), F32)] * 3,
      name="at11_phase_b3",
  )(xdom0, xdom1, cnt0, cnt1, w3, wv2, wlin)


_C1_ROWS = 80
_C1_CHUNKS = M // _C1_ROWS


def _phase_c1_body(tm, qt, rt, msd, mtd, z128,
                   a_out, y0_out, y1_out,
                   tm_v, q_v, r_v, a_v, cv_v, msd_v, mtd_v, zrow_v, sem, sem2,
                   acc_y):
  c = lax.axis_index("c")
  s = lax.axis_index("s")
  wid = c * 16 + s

  pltpu.sync_copy(z128, zrow_v)
  pltpu.sync_copy(zrow_v, acc_y.at[pl.ds(128 * s, 128)])
  plsc.subcore_barrier()

  def _chunk(k, carry):
    ch = wid + 32 * k
    @pl.when(ch < _C1_CHUNKS)
    def _():
      r0 = _C1_ROWS * ch
      pltpu.sync_copy(tm.at[pl.ds(r0, _C1_ROWS)], tm_v)
      pltpu.sync_copy(msd.at[pl.ds(r0, _C1_ROWS)], msd_v)
      pltpu.sync_copy(mtd.at[pl.ds(r0, _C1_ROWS)], mtd_v)
      pltpu.async_copy(qt.at[msd_v], q_v, sem)
      pltpu.async_copy(rt.at[msd_v], r_v, sem2)
      pltpu.make_async_copy(qt.at[msd_v], q_v, sem).wait()
      pltpu.make_async_copy(rt.at[msd_v], r_v, sem2).wait()
      def _rows(r, cc):
        for q in range(O // 16):
          sl = pl.ds(16 * q, 16)
          sl2 = pl.ds(O + 16 * q, 16)
          a_v[r, sl] = tm_v[r, sl] + q_v[r, sl]
          cv_v[r, sl] = tm_v[r, sl2] + r_v[r, sl]
        return cc
      lax.fori_loop(0, _C1_ROWS, _rows, 0)
      pltpu.sync_copy(a_v, a_out.at[pl.ds(r0, _C1_ROWS)])
      pltpu.sync_copy(cv_v, acc_y.at[mtd_v], add=True)
    return carry
  lax.fori_loop(0, (_C1_CHUNKS + 31) // 32, _chunk, 0)

  plsc.subcore_barrier()

  def _wb(y_out):
    @pl.when(s < 15)
    def _():
      pltpu.sync_copy(acc_y.at[pl.ds(128 * s, 128)],
                      y_out.at[pl.ds(128 * s, 128)])
    @pl.when(s == 15)
    def _():
      pltpu.sync_copy(acc_y.at[pl.ds(1920, D - 1920)],
                      y_out.at[pl.ds(1920, D - 1920)])

  @pl.when(c == 0)
  def _():
    _wb(y0_out)

  @pl.when(c == 1)
  def _():
    _wb(y1_out)


def _phase_c1(tm, qt, rt, msd, mtd, z128):
  return pl.kernel(
      _phase_c1_body,
      out_type=(
          jax.ShapeDtypeStruct((M + 96, O), F32),
          jax.ShapeDtypeStruct((D, O), F32),
          jax.ShapeDtypeStruct((D, O), F32),
      ),
      mesh=_mesh(),
      scratch_types=[
          pltpu.VMEM((_C1_ROWS, 2 * O), F32),
          pltpu.VMEM((_C1_ROWS, O), F32),
          pltpu.VMEM((_C1_ROWS, O), F32),
          pltpu.VMEM((_C1_ROWS, O), F32),
          pltpu.VMEM((_C1_ROWS, O), F32),
          pltpu.VMEM((_C1_ROWS,), I32),
          pltpu.VMEM((_C1_ROWS,), I32),
          pltpu.VMEM((128, C), F32),
          pltpu.SemaphoreType.DMA,
          pltpu.SemaphoreType.DMA,
          pltpu.VMEM_SHARED((2048, O), F32),
      ],
      name="at11_phase_c1",
  )(tm, qt, rt, msd, mtd, z128)


def _phase_c2_body(xw1, a_tab, srcp, iip, tgtp, z128,
                   out0, out1,
                   r1a, r1b, r2a, r2b, srcv0, srcv1, iiv0, iiv1, tgtv0, tgtv1,
                   zrow_v, g1s0, g1s1, g2s0, g2s1, s1s0, s1s1, s2s0, s2s1,
                   acc):
  c = lax.axis_index("c")
  s = lax.axis_index("s")

  pltpu.sync_copy(z128, zrow_v)

  def _z(k, carry):
    ch = s + 16 * k
    @pl.when(ch < _N_CHUNKS)
    def _():
      pltpu.sync_copy(zrow_v, acc.at[pl.ds(128 * ch, 128)])
    return carry
  lax.fori_loop(0, 5, _z, 0)

  plsc.subcore_barrier()

  base_ch = 640 * c + 40 * s
  bufs = ((r1a, r2a, srcv0, iiv0, tgtv0, g1s0, g2s0, s1s0, s2s0),
          (r1b, r2b, srcv1, iiv1, tgtv1, g1s1, g2s1, s1s1, s2s1))

  def _stage(b, ch):
    r1, r2, srcv, iiv, tgtv, g1, g2, _, _ = b
    p0 = 128 * ch
    pltpu.sync_copy(srcp.at[pl.ds(p0, 128)], srcv)
    pltpu.sync_copy(iip.at[pl.ds(p0, 128)], iiv)
    pltpu.sync_copy(tgtp.at[pl.ds(p0, 128)], tgtv)
    pltpu.async_copy(xw1.at[srcv], r1, g1)
    pltpu.async_copy(a_tab.at[iiv], r2, g2)

  def _wait_g(b):
    r1, r2, srcv, iiv, _, g1, g2, _, _ = b
    pltpu.make_async_copy(xw1.at[srcv], r1, g1).wait()
    pltpu.make_async_copy(a_tab.at[iiv], r2, g2).wait()

  def _scat(b):
    r1, r2, _, _, tgtv, _, _, s1, s2 = b
    idx = plsc.Indices(tgtv, ignored_value=-1)
    pltpu.sync_copy(r1, acc.at[idx], add=True)
    pltpu.sync_copy(r2, acc.at[idx], add=True)

  def _wait_s(b):
    pass

  def _chunk(k, carry):
    ch = base_ch + k
    b = bufs[0]
    _stage(b, ch)
    _wait_g(b)
    r1, r2, _, _, tgtv, _, _, s1, s2 = b
    pltpu.sync_copy(r1, acc.at[tgtv], add=True)
    pltpu.sync_copy(r2, acc.at[tgtv], add=True)
    return carry
  lax.fori_loop(0, 40, _chunk, 0)

  plsc.subcore_barrier()

  def _wb(out):
    def _w(k, carry):
      ch = s + 16 * k
      @pl.when(ch < _N_CHUNKS)
      def _():
        pltpu.sync_copy(acc.at[pl.ds(128 * ch, 128)],
                        out.at[pl.ds(128 * ch, 128)])
      return carry
    lax.fori_loop(0, 5, _w, 0)

  @pl.when(c == 0)
  def _():
    _wb(out0)

  @pl.when(c == 1)
  def _():
    _wb(out1)


def _phase_c2(xw1, a_tab, srcp, iip, tgtp, z128):
  return pl.kernel(
      _phase_c2_body,
      out_type=(
          jax.ShapeDtypeStruct((NPAD, O), F32),
          jax.ShapeDtypeStruct((NPAD, O), F32),
      ),
      mesh=_mesh(),
      scratch_types=[
          pltpu.VMEM((128, O), F32),
          pltpu.VMEM((128, O), F32),
          pltpu.VMEM((128, O), F32),
          pltpu.VMEM((128, O), F32),
          pltpu.VMEM((128,), I32),
          pltpu.VMEM((128,), I32),
          pltpu.VMEM((128,), I32),
          pltpu.VMEM((128,), I32),
          pltpu.VMEM((128,), I32),
          pltpu.VMEM((128,), I32),
          pltpu.VMEM((128, C), F32),
          pltpu.SemaphoreType.DMA,
          pltpu.SemaphoreType.DMA,
          pltpu.SemaphoreType.DMA,
          pltpu.SemaphoreType.DMA,
          pltpu.SemaphoreType.DMA,
          pltpu.SemaphoreType.DMA,
          pltpu.SemaphoreType.DMA,
          pltpu.SemaphoreType.DMA,
          pltpu.VMEM_SHARED((NPAD, O), F32),
      ],
      compiler_params=pltpu.CompilerParams(use_tc_tiling_on_sc=False),
      name="at11_phase_c2",
  )(xw1, a_tab, srcp, iip, tgtp, z128)


_D_ROWS = 80
_D_CHUNKS = N // _D_ROWS


def _phase_d_body(base, out0, out1, invm, y0, y1, domp,
                  outf,
                  domv, g1, g2, g3, b1, b2, b3, o_v, sem, sem2, sem3):
  c = lax.axis_index("c")
  s = lax.axis_index("s")
  wid = c * 16 + s

  def _chunk(k, carry):
    ch = wid + 32 * k
    @pl.when(ch < _D_CHUNKS)
    def _():
      r0 = _D_ROWS * ch
      pltpu.sync_copy(domp.at[pl.ds(r0, _D_ROWS)], domv)
      pltpu.async_copy(invm.at[domv], g1, sem)
      pltpu.async_copy(y0.at[domv], g2, sem2)
      pltpu.async_copy(y1.at[domv], g3, sem3)
      pltpu.sync_copy(base.at[pl.ds(r0, _D_ROWS)], b1)
      pltpu.sync_copy(out0.at[pl.ds(r0, _D_ROWS)], b2)
      pltpu.sync_copy(out1.at[pl.ds(r0, _D_ROWS)], b3)
      pltpu.make_async_copy(invm.at[domv], g1, sem).wait()
      pltpu.make_async_copy(y0.at[domv], g2, sem2).wait()
      pltpu.make_async_copy(y1.at[domv], g3, sem3).wait()
      def _rows(r, cc):
        for q in range(O // 16):
          sl = pl.ds(16 * q, 16)
          o_v[r, sl] = ((b1[r, sl] + b2[r, sl]) + (b3[r, sl] + g1[r, sl])
                        + (g2[r, sl] + g3[r, sl]))
        return cc
      lax.fori_loop(0, _D_ROWS, _rows, 0)
      pltpu.sync_copy(o_v, outf.at[pl.ds(r0, _D_ROWS)])
    return carry
  lax.fori_loop(0, (_D_CHUNKS + 31) // 32, _chunk, 0)


def _phase_d(base, out0, out1, invm, y0, y1, domp):
  return pl.kernel(
      _phase_d_body,
      out_type=jax.ShapeDtypeStruct((N, O), F32),
      mesh=_mesh(),
      scratch_types=[
          pltpu.VMEM((_D_ROWS,), I32),
          pltpu.VMEM((_D_ROWS, O), F32),
          pltpu.VMEM((_D_ROWS, O), F32),
          pltpu.VMEM((_D_ROWS, O), F32),
          pltpu.VMEM((_D_ROWS, O), F32),
          pltpu.VMEM((_D_ROWS, O), F32),
          pltpu.VMEM((_D_ROWS, O), F32),
          pltpu.VMEM((_D_ROWS, O), F32),
          pltpu.SemaphoreType.DMA,
          pltpu.SemaphoreType.DMA,
          pltpu.SemaphoreType.DMA,
      ],
      name="at11_phase_d",
  )(base, out0, out1, invm, y0, y1, domp)


def kernel(x, domain_indicator, source_row_idx, target_row_idx,
           intersect_indicator, msg_src_domain, msg_tgt_domain,
           W_tf_intersect, W_tf_invariant, W_linmaps_invariant, W_linmaps_id):
  x = x.astype(F32)
  xpad = jnp.concatenate([x, jnp.zeros((NPAD - N, C), F32)], axis=0)
  domp = jnp.concatenate(
      [domain_indicator.astype(I32), jnp.full((NPAD - N,), D, I32)])
  srcp = jnp.concatenate([source_row_idx.astype(I32), jnp.zeros((P2 - P,), I32)])
  tgtp = jnp.concatenate([target_row_idx.astype(I32), jnp.full((P2 - P,), N, I32)])
  iip = jnp.concatenate([intersect_indicator.astype(I32), jnp.full((P2 - P,), M, I32)])
  msd = msg_src_domain.astype(I32)
  mtd = msg_tgt_domain.astype(I32)

  w1 = W_tf_intersect[0:C]
  w2 = W_tf_intersect[C:2 * C]
  w3 = W_tf_intersect[2 * C:3 * C]
  wv1 = W_tf_invariant[0:C]
  wv2 = W_tf_invariant[C:2 * C]
  w_cat = jnp.concatenate([w2, wv1], axis=1)

  z128 = jnp.zeros((128, C), F32)
  zq = jnp.zeros((128, _CQ), F32)
  z16 = jnp.zeros((128, 16), F32)
  one16 = jnp.ones((128, 16), F32)
  xq = [xpad[:, _CQ * k:_CQ * (k + 1)] for k in range(4)]

  (mq0, mq1, mq2, mq3, xdom0, xdom1, cnt0, cnt1) = _phase_a(
      xq, xpad, srcp, iip, domp, z128, zq, z16, one16)
  base, xw1 = _phase_b1(x, W_linmaps_id, w1)
  tm = _phase_b2((mq0, mq1, mq2, mq3), w_cat)
  qt, rt, invm = _phase_b3(xdom0, xdom1, cnt0, cnt1, w3, wv2,
                           W_linmaps_invariant)
  a_tab, y0, y1 = _phase_c1(tm, qt, rt, msd, mtd, z128)
  out0, out1 = _phase_c2(xw1, a_tab, srcp, iip, tgtp, z128)
  return _phase_d(base, out0, out1, invm, y0, y1,
                  domain_indicator.astype(I32))


# speedup vs baseline: 1.0611x; 1.0611x over previous
import functools

import jax
import jax.numpy as jnp
from jax import lax
from jax.experimental import pallas as pl
from jax.experimental.pallas import tpu as pltpu
from jax.experimental.pallas import tpu_sc as plsc

N, C, O = 10000, 128, 128
D = 2000
M = 20000
P = 160000

NPAD = 10112
HALF_M = M // 2
HALF_D = D // 2
F32 = jnp.float32
I32 = jnp.int32

_N_CHUNKS = NPAD // 128
P2 = 163840
_P2_CHUNKS = P2 // 128


def _mesh():
  return plsc.VectorSubcoreMesh(core_axis_name="c", subcore_axis_name="s")


def _mask_local(vals_ref, idx_ref, base, size, n):
  for j in range(n // 16):
    sl = pl.ds(j * 16, 16)
    v = vals_ref[sl]
    loc = v - base
    ok = (loc >= 0) & (loc < size)
    idx_ref[sl] = jnp.where(ok, loc, -1)


_MROWS = 20096
_CQ = C // 4


def _phase_a_body(xq0, xq1, xq2, xq3, xpad, srcp, iip, domp, z128, zq, z16,
                  one16,
                  msgq0, msgq1, msgq2, msgq3,
                  xdom0_out, xdom1_out, cnt0_out, cnt1_out,
                  rows_v, rowq0, rowq1, zrow_v, zq_v, z16_v, one16_v,
                  srcv0, srcv1, iiv0, iiv1, gsem0, gsem1, ssem0, ssem1,
                  acc_msg, acc_dom, acc_cnt):
  c = lax.axis_index("c")
  s = lax.axis_index("s")

  pltpu.sync_copy(z128, zrow_v)
  pltpu.sync_copy(zq, zq_v)
  pltpu.sync_copy(z16, z16_v)
  pltpu.sync_copy(one16, one16_v)

  for rnd, (xq, msgq) in enumerate(
      ((xq0, msgq0), (xq1, msgq1), (xq2, msgq2), (xq3, msgq3))):
    def _zmsg(k, carry):
      ch = s + 16 * k
      @pl.when(ch < _MROWS // 128)
      def _():
        pltpu.sync_copy(zq_v, acc_msg.at[pl.ds(128 * ch, 128)])
      return carry
    lax.fori_loop(0, 10, _zmsg, 0)
    plsc.subcore_barrier()

    bufs = ((rowq0, srcv0, iiv0, gsem0, ssem0),
            (rowq1, srcv1, iiv1, gsem1, ssem1))

    def _stage(b, ch):
      rowq, srcv, iiv, gsem, _ = b
      p0 = 128 * ch
      pltpu.sync_copy(srcp.at[pl.ds(p0, 128)], srcv)
      pltpu.sync_copy(iip.at[pl.ds(p0, 128)], iiv)
      pltpu.async_copy(xq.at[srcv], rowq, gsem)

    def _wait_g(b):
      rowq, srcv, _, gsem, _ = b
      pltpu.make_async_copy(xq.at[srcv], rowq, gsem).wait()

    def _scat(b):
      rowq, _, iiv, _, ssem = b
      pltpu.async_copy(rowq, acc_msg.at[iiv], ssem, add=True)

    def _wait_s(b):
      rowq, _, iiv, _, ssem = b
      pltpu.make_async_copy(rowq, acc_msg.at[iiv], ssem).wait()

    def _chunk(k, carry):
      b = bufs[0]
      _stage(b, 640 * c + s + 16 * k)
      _wait_g(b)
      rowq, _, iiv, _, _ = b
      pltpu.sync_copy(rowq, acc_msg.at[iiv], add=True)
      return carry
    lax.fori_loop(0, 40, _chunk, 0)
    plsc.subcore_barrier()

    def _wmsg(k, carry):
      ch = s + 16 * k
      @pl.when(ch < M // 128)
      def _():
        pltpu.sync_copy(acc_msg.at[pl.ds(128 * ch, 128)],
                        msgq.at[c, pl.ds(128 * ch, 128)])
      return carry
    lax.fori_loop(0, 10, _wmsg, 0)

    @pl.when(s == 15)
    def _():
      t0 = (M // 128) * 128
      pltpu.sync_copy(acc_msg.at[pl.ds(t0, M - t0)],
                      msgq.at[c, pl.ds(t0, M - t0)])
    plsc.subcore_barrier()

  pltpu.sync_copy(zrow_v, acc_dom.at[pl.ds(128 * s, 128)])

  @pl.when(s == 8)
  def _():
    def _zc(k, carry):
      pltpu.sync_copy(z16_v, acc_cnt.at[pl.ds(128 * k, 128)])
      return carry
    lax.fori_loop(0, 16, _zc, 0)

  plsc.subcore_barrier()

  def _row(k, carry):
    ch = 40 * c + s + 16 * k
    @pl.when((ch < 40 * c + 40) & (ch < _N_CHUNKS))
    def _():
      r0 = 128 * ch
      pltpu.sync_copy(xpad.at[pl.ds(r0, 128)], rows_v)
      pltpu.sync_copy(domp.at[pl.ds(r0, 128)], srcv0)
      pltpu.sync_copy(rows_v, acc_dom.at[srcv0], add=True)
      pltpu.sync_copy(one16_v, acc_cnt.at[srcv0], add=True)
    return carry
  lax.fori_loop(0, 3, _row, 0)

  plsc.subcore_barrier()

  def _wdom(xdom_out, cnt_out):
    @pl.when(s < 15)
    def _():
      pltpu.sync_copy(acc_dom.at[pl.ds(128 * s, 128)],
                      xdom_out.at[pl.ds(128 * s, 128)])
    @pl.when(s == 15)
    def _():
      pltpu.sync_copy(acc_dom.at[pl.ds(1920, D - 1920)],
                      xdom_out.at[pl.ds(1920, D - 1920)])
      pltpu.sync_copy(acc_cnt.at[pl.ds(0, D)], cnt_out.at[pl.ds(0, D)])

  @pl.when(c == 0)
  def _():
    _wdom(xdom0_out, cnt0_out)

  @pl.when(c == 1)
  def _():
    _wdom(xdom1_out, cnt1_out)


def _phase_a(xq, xpad, srcp, iip, domp, z128, zq, z16, one16):
  return pl.kernel(
      _phase_a_body,
      out_type=(
          jax.ShapeDtypeStruct((2, M, _CQ), F32),
          jax.ShapeDtypeStruct((2, M, _CQ), F32),
          jax.ShapeDtypeStruct((2, M, _CQ), F32),
          jax.ShapeDtypeStruct((2, M, _CQ), F32),
          jax.ShapeDtypeStruct((D, C), F32),
          jax.ShapeDtypeStruct((D, C), F32),
          jax.ShapeDtypeStruct((D, 16), F32),
          jax.ShapeDtypeStruct((D, 16), F32),
      ),
      mesh=_mesh(),
      scratch_types=[
          pltpu.VMEM((128, C), F32),
          pltpu.VMEM((128, _CQ), F32),
          pltpu.VMEM((128, _CQ), F32),
          pltpu.VMEM((128, C), F32),
          pltpu.VMEM((128, _CQ), F32),
          pltpu.VMEM((128, 16), F32),
          pltpu.VMEM((128, 16), F32),
          pltpu.VMEM((128,), I32),
          pltpu.VMEM((128,), I32),
          pltpu.VMEM((128,), I32),
          pltpu.VMEM((128,), I32),
          pltpu.SemaphoreType.DMA,
          pltpu.SemaphoreType.DMA,
          pltpu.SemaphoreType.DMA,
          pltpu.SemaphoreType.DMA,
          pltpu.VMEM_SHARED((_MROWS, _CQ), F32),
          pltpu.VMEM_SHARED((2048, C), F32),
          pltpu.VMEM_SHARED((2048, 16), F32),
      ],
      compiler_params=pltpu.CompilerParams(use_tc_tiling_on_sc=False),
      name="at11_phase_a",
  )(xq[0], xq[1], xq[2], xq[3], xpad, srcp, iip, domp, z128, zq, z16, one16)


def _mm2_body(x_ref, wa_ref, wb_ref, oa_ref, ob_ref):
  xb = x_ref[...]
  oa_ref[...] = jnp.dot(xb, wa_ref[...], preferred_element_type=F32)
  ob_ref[...] = jnp.dot(xb, wb_ref[...], preferred_element_type=F32)


def _phase_b1(x, w_id, w1):
  return pl.pallas_call(
      _mm2_body,
      grid=(N // 400,),
      in_specs=[
          pl.BlockSpec((400, C), lambda i: (i, 0)),
          pl.BlockSpec((C, O), lambda i: (0, 0)),
          pl.BlockSpec((C, O), lambda i: (0, 0)),
      ],
      out_specs=[
          pl.BlockSpec((400, O), lambda i: (i, 0)),
          pl.BlockSpec((400, O), lambda i: (i, 0)),
      ],
      out_shape=[
          jax.ShapeDtypeStruct((N, O), F32),
          jax.ShapeDtypeStruct((N, O), F32),
      ],
      name="at11_phase_b1",
  )(x, w_id, w1)


def _b2_body(q0_ref, q1_ref, q2_ref, q3_ref, w_ref, o_ref):
  parts = []
  for q_ref in (q0_ref, q1_ref, q2_ref, q3_ref):
    q = q_ref[...]
    parts.append(q[0] + q[1])
  msg = jnp.concatenate(parts, axis=1)
  o_ref[...] = jnp.dot(msg, w_ref[...], preferred_element_type=F32)


def _phase_b2(msgq, w_cat):
  qspec = pl.BlockSpec((2, 400, _CQ), lambda i: (0, i, 0))
  return pl.pallas_call(
      _b2_body,
      grid=(M // 400,),
      in_specs=[qspec, qspec, qspec, qspec,
                pl.BlockSpec((C, 2 * O), lambda i: (0, 0))],
      out_specs=pl.BlockSpec((400, 2 * O), lambda i: (i, 0)),
      out_shape=jax.ShapeDtypeStruct((M, 2 * O), F32),
      name="at11_phase_b2",
  )(msgq[0], msgq[1], msgq[2], msgq[3], w_cat)


def _b3_body(sum0_ref, sum1_ref, cnt0_ref, cnt1_ref, w3_ref, wv2_ref, wlin_ref,
             q_ref, r_ref, inv_ref):
  cnt = cnt0_ref[...][:, 0:1] + cnt1_ref[...][:, 0:1]
  xd = (sum0_ref[...] + sum1_ref[...]) / jnp.maximum(cnt, 1.0)
  q_ref[...] = jnp.dot(xd, w3_ref[...], preferred_element_type=F32)
  r_ref[...] = jnp.dot(xd, wv2_ref[...], preferred_element_type=F32)
  inv_ref[...] = jnp.dot(xd, wlin_ref[...], preferred_element_type=F32)


def _phase_b3(xdom0, xdom1, cnt0, cnt1, w3, wv2, wlin):
  return pl.pallas_call(
      _b3_body,
      out_shape=[jax.ShapeDtypeStruct((D, O), F32)] * 3,
      name="at11_phase_b3",
  )(xdom0, xdom1, cnt0, cnt1, w3, wv2, wlin)


_C1_ROWS = 80
_C1_CHUNKS = M // _C1_ROWS


def _phase_c1_body(tm, qt, rt, msd, mtd, z128,
                   a_out, y0_out, y1_out,
                   tm_v, q_v, r_v, a_v, cv_v, msd_v, mtd_v, zrow_v, sem, sem2,
                   acc_y):
  c = lax.axis_index("c")
  s = lax.axis_index("s")
  wid = c * 16 + s

  pltpu.sync_copy(z128, zrow_v)
  pltpu.sync_copy(zrow_v, acc_y.at[pl.ds(128 * s, 128)])
  plsc.subcore_barrier()

  def _chunk(k, carry):
    ch = wid + 32 * k
    @pl.when(ch < _C1_CHUNKS)
    def _():
      r0 = _C1_ROWS * ch
      pltpu.sync_copy(tm.at[pl.ds(r0, _C1_ROWS)], tm_v)
      pltpu.sync_copy(msd.at[pl.ds(r0, _C1_ROWS)], msd_v)
      pltpu.sync_copy(mtd.at[pl.ds(r0, _C1_ROWS)], mtd_v)
      pltpu.async_copy(qt.at[msd_v], q_v, sem)
      pltpu.async_copy(rt.at[msd_v], r_v, sem2)
      pltpu.make_async_copy(qt.at[msd_v], q_v, sem).wait()
      pltpu.make_async_copy(rt.at[msd_v], r_v, sem2).wait()
      def _rows(r, cc):
        for q in range(O // 16):
          sl = pl.ds(16 * q, 16)
          sl2 = pl.ds(O + 16 * q, 16)
          a_v[r, sl] = tm_v[r, sl] + q_v[r, sl]
          cv_v[r, sl] = tm_v[r, sl2] + r_v[r, sl]
        return cc
      lax.fori_loop(0, _C1_ROWS, _rows, 0)
      pltpu.sync_copy(a_v, a_out.at[pl.ds(r0, _C1_ROWS)])
      pltpu.sync_copy(cv_v, acc_y.at[mtd_v], add=True)
    return carry
  lax.fori_loop(0, (_C1_CHUNKS + 31) // 32, _chunk, 0)

  plsc.subcore_barrier()

  def _wb(y_out):
    @pl.when(s < 15)
    def _():
      pltpu.sync_copy(acc_y.at[pl.ds(128 * s, 128)],
                      y_out.at[pl.ds(128 * s, 128)])
    @pl.when(s == 15)
    def _():
      pltpu.sync_copy(acc_y.at[pl.ds(1920, D - 1920)],
                      y_out.at[pl.ds(1920, D - 1920)])

  @pl.when(c == 0)
  def _():
    _wb(y0_out)

  @pl.when(c == 1)
  def _():
    _wb(y1_out)


def _phase_c1(tm, qt, rt, msd, mtd, z128):
  return pl.kernel(
      _phase_c1_body,
      out_type=(
          jax.ShapeDtypeStruct((M + 96, O), F32),
          jax.ShapeDtypeStruct((D, O), F32),
          jax.ShapeDtypeStruct((D, O), F32),
      ),
      mesh=_mesh(),
      scratch_types=[
          pltpu.VMEM((_C1_ROWS, 2 * O), F32),
          pltpu.VMEM((_C1_ROWS, O), F32),
          pltpu.VMEM((_C1_ROWS, O), F32),
          pltpu.VMEM((_C1_ROWS, O), F32),
          pltpu.VMEM((_C1_ROWS, O), F32),
          pltpu.VMEM((_C1_ROWS,), I32),
          pltpu.VMEM((_C1_ROWS,), I32),
          pltpu.VMEM((128, C), F32),
          pltpu.SemaphoreType.DMA,
          pltpu.SemaphoreType.DMA,
          pltpu.VMEM_SHARED((2048, O), F32),
      ],
      name="at11_phase_c1",
  )(tm, qt, rt, msd, mtd, z128)


def _phase_c2_body(xw1, a_tab, srcp, iip, tgtp, z128,
                   out0, out1,
                   r1a, r1b, r2a, r2b, srcv0, srcv1, iiv0, iiv1, tgtv0, tgtv1,
                   zrow_v, g1s0, g1s1, g2s0, g2s1, s1s0, s1s1, s2s0, s2s1,
                   acc):
  c = lax.axis_index("c")
  s = lax.axis_index("s")

  pltpu.sync_copy(z128, zrow_v)

  def _z(k, carry):
    ch = s + 16 * k
    @pl.when(ch < _N_CHUNKS)
    def _():
      pltpu.sync_copy(zrow_v, acc.at[pl.ds(128 * ch, 128)])
    return carry
  lax.fori_loop(0, 5, _z, 0)

  plsc.subcore_barrier()

  bufs = ((r1a, r2a, srcv0, iiv0, tgtv0, g1s0, g2s0, s1s0, s2s0),
          (r1b, r2b, srcv1, iiv1, tgtv1, g1s1, g2s1, s1s1, s2s1))

  def _stage(b, ch):
    r1, r2, srcv, iiv, tgtv, g1, g2, _, _ = b
    p0 = 128 * ch
    pltpu.sync_copy(srcp.at[pl.ds(p0, 128)], srcv)
    pltpu.sync_copy(iip.at[pl.ds(p0, 128)], iiv)
    pltpu.sync_copy(tgtp.at[pl.ds(p0, 128)], tgtv)
    pltpu.async_copy(xw1.at[srcv], r1, g1)
    pltpu.async_copy(a_tab.at[iiv], r2, g2)

  def _wait_g(b):
    r1, r2, srcv, iiv, _, g1, g2, _, _ = b
    pltpu.make_async_copy(xw1.at[srcv], r1, g1).wait()
    pltpu.make_async_copy(a_tab.at[iiv], r2, g2).wait()

  def _scat(b):
    r1, r2, _, _, tgtv, _, _, s1, s2 = b
    idx = plsc.Indices(tgtv, ignored_value=-1)
    pltpu.sync_copy(r1, acc.at[idx], add=True)
    pltpu.sync_copy(r2, acc.at[idx], add=True)

  def _wait_s(b):
    pass

  def _chunk(k, carry):
    ch = 640 * c + s + 16 * k
    b = bufs[0]
    _stage(b, ch)
    _wait_g(b)
    r1, r2, _, _, tgtv, _, _, s1, s2 = b
    pltpu.async_copy(r1, acc.at[tgtv], s1, add=True)
    pltpu.async_copy(r2, acc.at[tgtv], s2, add=True)
    pltpu.make_async_copy(r1, acc.at[tgtv], s1).wait()
    pltpu.make_async_copy(r2, acc.at[tgtv], s2).wait()
    return carry
  lax.fori_loop(0, 40, _chunk, 0)

  plsc.subcore_barrier()

  def _wb(out):
    def _w(k, carry):
      ch = s + 16 * k
      @pl.when(ch < _N_CHUNKS)
      def _():
        pltpu.sync_copy(acc.at[pl.ds(128 * ch, 128)],
                        out.at[pl.ds(128 * ch, 128)])
      return carry
    lax.fori_loop(0, 5, _w, 0)

  @pl.when(c == 0)
  def _():
    _wb(out0)

  @pl.when(c == 1)
  def _():
    _wb(out1)


def _phase_c2(xw1, a_tab, srcp, iip, tgtp, z128):
  return pl.kernel(
      _phase_c2_body,
      out_type=(
          jax.ShapeDtypeStruct((NPAD, O), F32),
          jax.ShapeDtypeStruct((NPAD, O), F32),
      ),
      mesh=_mesh(),
      scratch_types=[
          pltpu.VMEM((128, O), F32),
          pltpu.VMEM((128, O), F32),
          pltpu.VMEM((128, O), F32),
          pltpu.VMEM((128, O), F32),
          pltpu.VMEM((128,), I32),
          pltpu.VMEM((128,), I32),
          pltpu.VMEM((128,), I32),
          pltpu.VMEM((128,), I32),
          pltpu.VMEM((128,), I32),
          pltpu.VMEM((128,), I32),
          pltpu.VMEM((128, C), F32),
          pltpu.SemaphoreType.DMA,
          pltpu.SemaphoreType.DMA,
          pltpu.SemaphoreType.DMA,
          pltpu.SemaphoreType.DMA,
          pltpu.SemaphoreType.DMA,
          pltpu.SemaphoreType.DMA,
          pltpu.SemaphoreType.DMA,
          pltpu.SemaphoreType.DMA,
          pltpu.VMEM_SHARED((NPAD, O), F32),
      ],
      compiler_params=pltpu.CompilerParams(use_tc_tiling_on_sc=False),
      name="at11_phase_c2",
  )(xw1, a_tab, srcp, iip, tgtp, z128)


_D_ROWS = 80
_D_CHUNKS = N // _D_ROWS


def _phase_d_body(base, out0, out1, invm, y0, y1, domp,
                  outf,
                  domv, g1, g2, g3, b1, b2, b3, o_v, sem, sem2, sem3):
  c = lax.axis_index("c")
  s = lax.axis_index("s")
  wid = c * 16 + s

  def _chunk(k, carry):
    ch = wid + 32 * k
    @pl.when(ch < _D_CHUNKS)
    def _():
      r0 = _D_ROWS * ch
      pltpu.sync_copy(domp.at[pl.ds(r0, _D_ROWS)], domv)
      pltpu.async_copy(invm.at[domv], g1, sem)
      pltpu.async_copy(y0.at[domv], g2, sem2)
      pltpu.async_copy(y1.at[domv], g3, sem3)
      pltpu.sync_copy(base.at[pl.ds(r0, _D_ROWS)], b1)
      pltpu.sync_copy(out0.at[pl.ds(r0, _D_ROWS)], b2)
      pltpu.sync_copy(out1.at[pl.ds(r0, _D_ROWS)], b3)
      pltpu.make_async_copy(invm.at[domv], g1, sem).wait()
      pltpu.make_async_copy(y0.at[domv], g2, sem2).wait()
      pltpu.make_async_copy(y1.at[domv], g3, sem3).wait()
      def _rows(r, cc):
        for q in range(O // 16):
          sl = pl.ds(16 * q, 16)
          o_v[r, sl] = ((b1[r, sl] + b2[r, sl]) + (b3[r, sl] + g1[r, sl])
                        + (g2[r, sl] + g3[r, sl]))
        return cc
      lax.fori_loop(0, _D_ROWS, _rows, 0)
      pltpu.sync_copy(o_v, outf.at[pl.ds(r0, _D_ROWS)])
    return carry
  lax.fori_loop(0, (_D_CHUNKS + 31) // 32, _chunk, 0)


def _phase_d(base, out0, out1, invm, y0, y1, domp):
  return pl.kernel(
      _phase_d_body,
      out_type=jax.ShapeDtypeStruct((N, O), F32),
      mesh=_mesh(),
      scratch_types=[
          pltpu.VMEM((_D_ROWS,), I32),
          pltpu.VMEM((_D_ROWS, O), F32),
          pltpu.VMEM((_D_ROWS, O), F32),
          pltpu.VMEM((_D_ROWS, O), F32),
          pltpu.VMEM((_D_ROWS, O), F32),
          pltpu.VMEM((_D_ROWS, O), F32),
          pltpu.VMEM((_D_ROWS, O), F32),
          pltpu.VMEM((_D_ROWS, O), F32),
          pltpu.SemaphoreType.DMA,
          pltpu.SemaphoreType.DMA,
          pltpu.SemaphoreType.DMA,
      ],
      name="at11_phase_d",
  )(base, out0, out1, invm, y0, y1, domp)


def kernel(x, domain_indicator, source_row_idx, target_row_idx,
           intersect_indicator, msg_src_domain, msg_tgt_domain,
           W_tf_intersect, W_tf_invariant, W_linmaps_invariant, W_linmaps_id):
  x = x.astype(F32)
  xpad = jnp.concatenate([x, jnp.zeros((NPAD - N, C), F32)], axis=0)
  domp = jnp.concatenate(
      [domain_indicator.astype(I32), jnp.full((NPAD - N,), D, I32)])
  srcp = jnp.concatenate([source_row_idx.astype(I32), jnp.zeros((P2 - P,), I32)])
  tgtp = jnp.concatenate([target_row_idx.astype(I32), jnp.full((P2 - P,), N, I32)])
  iip = jnp.concatenate([intersect_indicator.astype(I32), jnp.full((P2 - P,), M, I32)])
  msd = msg_src_domain.astype(I32)
  mtd = msg_tgt_domain.astype(I32)

  w1 = W_tf_intersect[0:C]
  w2 = W_tf_intersect[C:2 * C]
  w3 = W_tf_intersect[2 * C:3 * C]
  wv1 = W_tf_invariant[0:C]
  wv2 = W_tf_invariant[C:2 * C]
  w_cat = jnp.concatenate([w2, wv1], axis=1)

  z128 = jnp.zeros((128, C), F32)
  zq = jnp.zeros((128, _CQ), F32)
  z16 = jnp.zeros((128, 16), F32)
  one16 = jnp.ones((128, 16), F32)
  xq = [xpad[:, _CQ * k:_CQ * (k + 1)] for k in range(4)]

  (mq0, mq1, mq2, mq3, xdom0, xdom1, cnt0, cnt1) = _phase_a(
      xq, xpad, srcp, iip, domp, z128, zq, z16, one16)
  base, xw1 = _phase_b1(x, W_linmaps_id, w1)
  tm = _phase_b2((mq0, mq1, mq2, mq3), w_cat)
  qt, rt, invm = _phase_b3(xdom0, xdom1, cnt0, cnt1, w3, wv2,
                           W_linmaps_invariant)
  a_tab, y0, y1 = _phase_c1(tm, qt, rt, msd, mtd, z128)
  out0, out1 = _phase_c2(xw1, a_tab, srcp, iip, tgtp, z128)
  return _phase_d(base, out0, out1, invm, y0, y1,
                  domain_indicator.astype(I32))


# speedup vs baseline: 1.1903x; 1.1218x over previous
import functools

import jax
import jax.numpy as jnp
from jax import lax
from jax.experimental import pallas as pl
from jax.experimental.pallas import tpu as pltpu
from jax.experimental.pallas import tpu_sc as plsc

N, C, O = 10000, 128, 128
D = 2000
M = 20000
P = 160000

NPAD = 10112
HALF_M = M // 2
HALF_D = D // 2
F32 = jnp.float32
I32 = jnp.int32

_N_CHUNKS = NPAD // 128
P2 = 163840
_P2_CHUNKS = P2 // 128


def _mesh():
  return plsc.VectorSubcoreMesh(core_axis_name="c", subcore_axis_name="s")


def _mask_local(vals_ref, idx_ref, base, size, n):
  for j in range(n // 16):
    sl = pl.ds(j * 16, 16)
    v = vals_ref[sl]
    loc = v - base
    ok = (loc >= 0) & (loc < size)
    idx_ref[sl] = jnp.where(ok, loc, -1)


_MROWS = 20096
_CQ = C // 4


def _phase_a_body(xq0, xq1, xq2, xq3, xpad, srcp, iip, domp, z128, zq, z16,
                  one16,
                  msgq0, msgq1, msgq2, msgq3,
                  xdom0_out, xdom1_out, cnt0_out, cnt1_out,
                  rows_v, rowq0, rowq1, zrow_v, zq_v, z16_v, one16_v,
                  srcv0, srcv1, iiv0, iiv1, gsem0, gsem1, ssem0, ssem1,
                  acc_msg, acc_dom, acc_cnt):
  c = lax.axis_index("c")
  s = lax.axis_index("s")

  pltpu.sync_copy(z128, zrow_v)
  pltpu.sync_copy(zq, zq_v)
  pltpu.sync_copy(z16, z16_v)
  pltpu.sync_copy(one16, one16_v)

  for rnd, (xq, msgq) in enumerate(
      ((xq0, msgq0), (xq1, msgq1), (xq2, msgq2), (xq3, msgq3))):
    def _zmsg(k, carry):
      ch = s + 16 * k
      @pl.when(ch < _MROWS // 128)
      def _():
        pltpu.sync_copy(zq_v, acc_msg.at[pl.ds(128 * ch, 128)])
      return carry
    lax.fori_loop(0, 10, _zmsg, 0)
    plsc.subcore_barrier()

    half = 625

    def _pair(k, carry):
      ch = half * c + s + 16 * k
      @pl.when(ch < half * (c + 1))
      def _():
        p0 = 128 * ch
        pltpu.sync_copy(srcp.at[pl.ds(p0, 128)], srcv0)
        pltpu.sync_copy(iip.at[pl.ds(p0, 128)], iiv0)
        pltpu.async_copy(xq.at[srcv0], rowq0, gsem0).wait()
        pltpu.sync_copy(rowq0, acc_msg.at[iiv0], add=True)
      return carry
    lax.fori_loop(0, 40, _pair, 0)
    plsc.subcore_barrier()

    def _wmsg(k, carry):
      ch = s + 16 * k
      @pl.when(ch < M // 128)
      def _():
        pltpu.sync_copy(acc_msg.at[pl.ds(128 * ch, 128)],
                        msgq.at[c, pl.ds(128 * ch, 128)])
      return carry
    lax.fori_loop(0, 10, _wmsg, 0)

    @pl.when(s == 15)
    def _():
      t0 = (M // 128) * 128
      pltpu.sync_copy(acc_msg.at[pl.ds(t0, M - t0)],
                      msgq.at[c, pl.ds(t0, M - t0)])
    plsc.subcore_barrier()

  pltpu.sync_copy(zrow_v, acc_dom.at[pl.ds(128 * s, 128)])

  @pl.when(s == 8)
  def _():
    def _zc(k, carry):
      pltpu.sync_copy(z16_v, acc_cnt.at[pl.ds(128 * k, 128)])
      return carry
    lax.fori_loop(0, 16, _zc, 0)

  plsc.subcore_barrier()

  def _row(k, carry):
    ch = 40 * c + s + 16 * k
    @pl.when((ch < 40 * c + 40) & (ch < _N_CHUNKS))
    def _():
      r0 = 128 * ch
      pltpu.sync_copy(xpad.at[pl.ds(r0, 128)], rows_v)
      pltpu.sync_copy(domp.at[pl.ds(r0, 128)], srcv0)
      pltpu.sync_copy(rows_v, acc_dom.at[srcv0], add=True)
      pltpu.sync_copy(one16_v, acc_cnt.at[srcv0], add=True)
    return carry
  lax.fori_loop(0, 3, _row, 0)

  plsc.subcore_barrier()

  def _wdom(xdom_out, cnt_out):
    @pl.when(s < 15)
    def _():
      pltpu.sync_copy(acc_dom.at[pl.ds(128 * s, 128)],
                      xdom_out.at[pl.ds(128 * s, 128)])
    @pl.when(s == 15)
    def _():
      pltpu.sync_copy(acc_dom.at[pl.ds(1920, D - 1920)],
                      xdom_out.at[pl.ds(1920, D - 1920)])
      pltpu.sync_copy(acc_cnt.at[pl.ds(0, D)], cnt_out.at[pl.ds(0, D)])

  @pl.when(c == 0)
  def _():
    _wdom(xdom0_out, cnt0_out)

  @pl.when(c == 1)
  def _():
    _wdom(xdom1_out, cnt1_out)


def _phase_a(xq, xpad, srcp, iip, domp, z128, zq, z16, one16):
  return pl.kernel(
      _phase_a_body,
      out_type=(
          jax.ShapeDtypeStruct((2, M, _CQ), F32),
          jax.ShapeDtypeStruct((2, M, _CQ), F32),
          jax.ShapeDtypeStruct((2, M, _CQ), F32),
          jax.ShapeDtypeStruct((2, M, _CQ), F32),
          jax.ShapeDtypeStruct((D, C), F32),
          jax.ShapeDtypeStruct((D, C), F32),
          jax.ShapeDtypeStruct((D, 16), F32),
          jax.ShapeDtypeStruct((D, 16), F32),
      ),
      mesh=_mesh(),
      scratch_types=[
          pltpu.VMEM((128, C), F32),
          pltpu.VMEM((128, _CQ), F32),
          pltpu.VMEM((128, _CQ), F32),
          pltpu.VMEM((128, C), F32),
          pltpu.VMEM((128, _CQ), F32),
          pltpu.VMEM((128, 16), F32),
          pltpu.VMEM((128, 16), F32),
          pltpu.VMEM((128,), I32),
          pltpu.VMEM((128,), I32),
          pltpu.VMEM((128,), I32),
          pltpu.VMEM((128,), I32),
          pltpu.SemaphoreType.DMA,
          pltpu.SemaphoreType.DMA,
          pltpu.SemaphoreType.DMA,
          pltpu.SemaphoreType.DMA,
          pltpu.VMEM_SHARED((_MROWS, _CQ), F32),
          pltpu.VMEM_SHARED((2048, C), F32),
          pltpu.VMEM_SHARED((2048, 16), F32),
      ],
      compiler_params=pltpu.CompilerParams(use_tc_tiling_on_sc=False),
      name="at11_phase_a",
  )(xq[0], xq[1], xq[2], xq[3], xpad, srcp, iip, domp, z128, zq, z16, one16)


def _mm2_body(x_ref, wa_ref, wb_ref, oa_ref, ob_ref):
  xb = x_ref[...]
  oa_ref[...] = jnp.dot(xb, wa_ref[...], preferred_element_type=F32)
  ob_ref[...] = jnp.dot(xb, wb_ref[...], preferred_element_type=F32)


def _phase_b1(x, w_id, w1):
  return pl.pallas_call(
      _mm2_body,
      grid=(N // 400,),
      in_specs=[
          pl.BlockSpec((400, C), lambda i: (i, 0)),
          pl.BlockSpec((C, O), lambda i: (0, 0)),
          pl.BlockSpec((C, O), lambda i: (0, 0)),
      ],
      out_specs=[
          pl.BlockSpec((400, O), lambda i: (i, 0)),
          pl.BlockSpec((400, O), lambda i: (i, 0)),
      ],
      out_shape=[
          jax.ShapeDtypeStruct((N, O), F32),
          jax.ShapeDtypeStruct((N, O), F32),
      ],
      name="at11_phase_b1",
  )(x, w_id, w1)


def _b2_body(q0_ref, q1_ref, q2_ref, q3_ref, w_ref, o_ref):
  parts = []
  for q_ref in (q0_ref, q1_ref, q2_ref, q3_ref):
    q = q_ref[...]
    parts.append(q[0] + q[1])
  msg = jnp.concatenate(parts, axis=1)
  o_ref[...] = jnp.dot(msg, w_ref[...], preferred_element_type=F32)


def _phase_b2(msgq, w_cat):
  qspec = pl.BlockSpec((2, 400, _CQ), lambda i: (0, i, 0))
  return pl.pallas_call(
      _b2_body,
      grid=(M // 400,),
      in_specs=[qspec, qspec, qspec, qspec,
                pl.BlockSpec((C, 2 * O), lambda i: (0, 0))],
      out_specs=pl.BlockSpec((400, 2 * O), lambda i: (i, 0)),
      out_shape=jax.ShapeDtypeStruct((M, 2 * O), F32),
      name="at11_phase_b2",
  )(msgq[0], msgq[1], msgq[2], msgq[3], w_cat)


def _b3_body(sum0_ref, sum1_ref, cnt0_ref, cnt1_ref, w3_ref, wv2_ref, wlin_ref,
             q_ref, r_ref, inv_ref):
  cnt = cnt0_ref[...][:, 0:1] + cnt1_ref[...][:, 0:1]
  xd = (sum0_ref[...] + sum1_ref[...]) / jnp.maximum(cnt, 1.0)
  q_ref[...] = jnp.dot(xd, w3_ref[...], preferred_element_type=F32)
  r_ref[...] = jnp.dot(xd, wv2_ref[...], preferred_element_type=F32)
  inv_ref[...] = jnp.dot(xd, wlin_ref[...], preferred_element_type=F32)


def _phase_b3(xdom0, xdom1, cnt0, cnt1, w3, wv2, wlin):
  return pl.pallas_call(
      _b3_body,
      out_shape=[jax.ShapeDtypeStruct((D, O), F32)] * 3,
      name="at11_phase_b3",
  )(xdom0, xdom1, cnt0, cnt1, w3, wv2, wlin)


_C1_ROWS = 80
_C1_CHUNKS = M // _C1_ROWS


def _phase_c1_body(tm, qt, rt, msd, mtd, z128,
                   a_out, y0_out, y1_out,
                   tm_v, q_v, r_v, a_v, cv_v, msd_v, mtd_v, zrow_v, sem, sem2,
                   acc_y):
  c = lax.axis_index("c")
  s = lax.axis_index("s")
  wid = c * 16 + s

  pltpu.sync_copy(z128, zrow_v)
  pltpu.sync_copy(zrow_v, acc_y.at[pl.ds(128 * s, 128)])
  plsc.subcore_barrier()

  def _chunk(k, carry):
    ch = wid + 32 * k
    @pl.when(ch < _C1_CHUNKS)
    def _():
      r0 = _C1_ROWS * ch
      pltpu.sync_copy(tm.at[pl.ds(r0, _C1_ROWS)], tm_v)
      pltpu.sync_copy(msd.at[pl.ds(r0, _C1_ROWS)], msd_v)
      pltpu.sync_copy(mtd.at[pl.ds(r0, _C1_ROWS)], mtd_v)
      pltpu.async_copy(qt.at[msd_v], q_v, sem)
      pltpu.async_copy(rt.at[msd_v], r_v, sem2)
      pltpu.make_async_copy(qt.at[msd_v], q_v, sem).wait()
      pltpu.make_async_copy(rt.at[msd_v], r_v, sem2).wait()
      def _rows(r, cc):
        for q in range(O // 16):
          sl = pl.ds(16 * q, 16)
          sl2 = pl.ds(O + 16 * q, 16)
          a_v[r, sl] = tm_v[r, sl] + q_v[r, sl]
          cv_v[r, sl] = tm_v[r, sl2] + r_v[r, sl]
        return cc
      lax.fori_loop(0, _C1_ROWS, _rows, 0)
      pltpu.sync_copy(a_v, a_out.at[pl.ds(r0, _C1_ROWS)])
      pltpu.sync_copy(cv_v, acc_y.at[mtd_v], add=True)
    return carry
  lax.fori_loop(0, (_C1_CHUNKS + 31) // 32, _chunk, 0)

  plsc.subcore_barrier()

  def _wb(y_out):
    @pl.when(s < 15)
    def _():
      pltpu.sync_copy(acc_y.at[pl.ds(128 * s, 128)],
                      y_out.at[pl.ds(128 * s, 128)])
    @pl.when(s == 15)
    def _():
      pltpu.sync_copy(acc_y.at[pl.ds(1920, D - 1920)],
                      y_out.at[pl.ds(1920, D - 1920)])

  @pl.when(c == 0)
  def _():
    _wb(y0_out)

  @pl.when(c == 1)
  def _():
    _wb(y1_out)


def _phase_c1(tm, qt, rt, msd, mtd, z128):
  return pl.kernel(
      _phase_c1_body,
      out_type=(
          jax.ShapeDtypeStruct((M + 96, O), F32),
          jax.ShapeDtypeStruct((D, O), F32),
          jax.ShapeDtypeStruct((D, O), F32),
      ),
      mesh=_mesh(),
      scratch_types=[
          pltpu.VMEM((_C1_ROWS, 2 * O), F32),
          pltpu.VMEM((_C1_ROWS, O), F32),
          pltpu.VMEM((_C1_ROWS, O), F32),
          pltpu.VMEM((_C1_ROWS, O), F32),
          pltpu.VMEM((_C1_ROWS, O), F32),
          pltpu.VMEM((_C1_ROWS,), I32),
          pltpu.VMEM((_C1_ROWS,), I32),
          pltpu.VMEM((128, C), F32),
          pltpu.SemaphoreType.DMA,
          pltpu.SemaphoreType.DMA,
          pltpu.VMEM_SHARED((2048, O), F32),
      ],
      name="at11_phase_c1",
  )(tm, qt, rt, msd, mtd, z128)


def _phase_c2_body(xw1, a_tab, srcp, iip, tgtp, z128,
                   out0, out1,
                   r1a, r1b, r2a, r2b, srcv0, srcv1, iiv0, iiv1, tgtv0, tgtv1,
                   zrow_v, g1s0, g1s1, g2s0, g2s1, s1s0, s1s1, s2s0, s2s1,
                   acc):
  c = lax.axis_index("c")
  s = lax.axis_index("s")

  pltpu.sync_copy(z128, zrow_v)

  def _z(k, carry):
    ch = s + 16 * k
    @pl.when(ch < _N_CHUNKS)
    def _():
      pltpu.sync_copy(zrow_v, acc.at[pl.ds(128 * ch, 128)])
    return carry
  lax.fori_loop(0, 5, _z, 0)

  plsc.subcore_barrier()

  bufs = ((r1a, r2a, srcv0, iiv0, tgtv0, g1s0, g2s0, s1s0, s2s0),
          (r1b, r2b, srcv1, iiv1, tgtv1, g1s1, g2s1, s1s1, s2s1))

  def _stage(b, ch):
    r1, r2, srcv, iiv, tgtv, g1, g2, _, _ = b
    p0 = 128 * ch
    pltpu.sync_copy(srcp.at[pl.ds(p0, 128)], srcv)
    pltpu.sync_copy(iip.at[pl.ds(p0, 128)], iiv)
    pltpu.sync_copy(tgtp.at[pl.ds(p0, 128)], tgtv)
    pltpu.async_copy(xw1.at[srcv], r1, g1)
    pltpu.async_copy(a_tab.at[iiv], r2, g2)

  def _wait_g(b):
    r1, r2, srcv, iiv, _, g1, g2, _, _ = b
    pltpu.make_async_copy(xw1.at[srcv], r1, g1).wait()
    pltpu.make_async_copy(a_tab.at[iiv], r2, g2).wait()

  def _scat(b):
    r1, r2, _, _, tgtv, _, _, s1, s2 = b
    idx = plsc.Indices(tgtv, ignored_value=-1)
    pltpu.sync_copy(r1, acc.at[idx], add=True)
    pltpu.sync_copy(r2, acc.at[idx], add=True)

  def _wait_s(b):
    pass

  def _chunk(k, carry):
    ch = 640 * c + s + 16 * k
    b = bufs[0]
    _stage(b, ch)
    _wait_g(b)
    r1, r2, _, _, tgtv, _, _, s1, s2 = b
    pltpu.async_copy(r1, acc.at[tgtv], s1, add=True)
    pltpu.async_copy(r2, acc.at[tgtv], s2, add=True)
    pltpu.make_async_copy(r1, acc.at[tgtv], s1).wait()
    pltpu.make_async_copy(r2, acc.at[tgtv], s2).wait()
    return carry
  lax.fori_loop(0, 40, _chunk, 0)

  plsc.subcore_barrier()

  def _wb(out):
    def _w(k, carry):
      ch = s + 16 * k
      @pl.when(ch < _N_CHUNKS)
      def _():
        pltpu.sync_copy(acc.at[pl.ds(128 * ch, 128)],
                        out.at[pl.ds(128 * ch, 128)])
      return carry
    lax.fori_loop(0, 5, _w, 0)

  @pl.when(c == 0)
  def _():
    _wb(out0)

  @pl.when(c == 1)
  def _():
    _wb(out1)


def _phase_c2(xw1, a_tab, srcp, iip, tgtp, z128):
  return pl.kernel(
      _phase_c2_body,
      out_type=(
          jax.ShapeDtypeStruct((NPAD, O), F32),
          jax.ShapeDtypeStruct((NPAD, O), F32),
      ),
      mesh=_mesh(),
      scratch_types=[
          pltpu.VMEM((128, O), F32),
          pltpu.VMEM((128, O), F32),
          pltpu.VMEM((128, O), F32),
          pltpu.VMEM((128, O), F32),
          pltpu.VMEM((128,), I32),
          pltpu.VMEM((128,), I32),
          pltpu.VMEM((128,), I32),
          pltpu.VMEM((128,), I32),
          pltpu.VMEM((128,), I32),
          pltpu.VMEM((128,), I32),
          pltpu.VMEM((128, C), F32),
          pltpu.SemaphoreType.DMA,
          pltpu.SemaphoreType.DMA,
          pltpu.SemaphoreType.DMA,
          pltpu.SemaphoreType.DMA,
          pltpu.SemaphoreType.DMA,
          pltpu.SemaphoreType.DMA,
          pltpu.SemaphoreType.DMA,
          pltpu.SemaphoreType.DMA,
          pltpu.VMEM_SHARED((NPAD, O), F32),
      ],
      compiler_params=pltpu.CompilerParams(use_tc_tiling_on_sc=False),
      name="at11_phase_c2",
  )(xw1, a_tab, srcp, iip, tgtp, z128)


_D_ROWS = 80
_D_CHUNKS = N // _D_ROWS


def _phase_d_body(base, out0, out1, invm, y0, y1, domp,
                  outf,
                  domv, g1, g2, g3, b1, b2, b3, o_v, sem, sem2, sem3):
  c = lax.axis_index("c")
  s = lax.axis_index("s")
  wid = c * 16 + s

  def _chunk(k, carry):
    ch = wid + 32 * k
    @pl.when(ch < _D_CHUNKS)
    def _():
      r0 = _D_ROWS * ch
      pltpu.sync_copy(domp.at[pl.ds(r0, _D_ROWS)], domv)
      pltpu.async_copy(invm.at[domv], g1, sem)
      pltpu.async_copy(y0.at[domv], g2, sem2)
      pltpu.async_copy(y1.at[domv], g3, sem3)
      pltpu.sync_copy(base.at[pl.ds(r0, _D_ROWS)], b1)
      pltpu.sync_copy(out0.at[pl.ds(r0, _D_ROWS)], b2)
      pltpu.sync_copy(out1.at[pl.ds(r0, _D_ROWS)], b3)
      pltpu.make_async_copy(invm.at[domv], g1, sem).wait()
      pltpu.make_async_copy(y0.at[domv], g2, sem2).wait()
      pltpu.make_async_copy(y1.at[domv], g3, sem3).wait()
      def _rows(r, cc):
        for q in range(O // 16):
          sl = pl.ds(16 * q, 16)
          o_v[r, sl] = ((b1[r, sl] + b2[r, sl]) + (b3[r, sl] + g1[r, sl])
                        + (g2[r, sl] + g3[r, sl]))
        return cc
      lax.fori_loop(0, _D_ROWS, _rows, 0)
      pltpu.sync_copy(o_v, outf.at[pl.ds(r0, _D_ROWS)])
    return carry
  lax.fori_loop(0, (_D_CHUNKS + 31) // 32, _chunk, 0)


def _phase_d(base, out0, out1, invm, y0, y1, domp):
  return pl.kernel(
      _phase_d_body,
      out_type=jax.ShapeDtypeStruct((N, O), F32),
      mesh=_mesh(),
      scratch_types=[
          pltpu.VMEM((_D_ROWS,), I32),
          pltpu.VMEM((_D_ROWS, O), F32),
          pltpu.VMEM((_D_ROWS, O), F32),
          pltpu.VMEM((_D_ROWS, O), F32),
          pltpu.VMEM((_D_ROWS, O), F32),
          pltpu.VMEM((_D_ROWS, O), F32),
          pltpu.VMEM((_D_ROWS, O), F32),
          pltpu.VMEM((_D_ROWS, O), F32),
          pltpu.SemaphoreType.DMA,
          pltpu.SemaphoreType.DMA,
          pltpu.SemaphoreType.DMA,
      ],
      name="at11_phase_d",
  )(base, out0, out1, invm, y0, y1, domp)


def kernel(x, domain_indicator, source_row_idx, target_row_idx,
           intersect_indicator, msg_src_domain, msg_tgt_domain,
           W_tf_intersect, W_tf_invariant, W_linmaps_invariant, W_linmaps_id):
  x = x.astype(F32)
  xpad = jnp.concatenate([x, jnp.zeros((NPAD - N, C), F32)], axis=0)
  domp = jnp.concatenate(
      [domain_indicator.astype(I32), jnp.full((NPAD - N,), D, I32)])
  srcp = jnp.concatenate([source_row_idx.astype(I32), jnp.zeros((P2 - P,), I32)])
  tgtp = jnp.concatenate([target_row_idx.astype(I32), jnp.full((P2 - P,), N, I32)])
  iip = jnp.concatenate([intersect_indicator.astype(I32), jnp.full((P2 - P,), M, I32)])
  msd = msg_src_domain.astype(I32)
  mtd = msg_tgt_domain.astype(I32)

  w1 = W_tf_intersect[0:C]
  w2 = W_tf_intersect[C:2 * C]
  w3 = W_tf_intersect[2 * C:3 * C]
  wv1 = W_tf_invariant[0:C]
  wv2 = W_tf_invariant[C:2 * C]
  w_cat = jnp.concatenate([w2, wv1], axis=1)

  z128 = jnp.zeros((128, C), F32)
  zq = jnp.zeros((128, _CQ), F32)
  z16 = jnp.zeros((128, 16), F32)
  one16 = jnp.ones((128, 16), F32)
  xq = [xpad[:, _CQ * k:_CQ * (k + 1)] for k in range(4)]

  (mq0, mq1, mq2, mq3, xdom0, xdom1, cnt0, cnt1) = _phase_a(
      xq, xpad, srcp, iip, domp, z128, zq, z16, one16)
  base, xw1 = _phase_b1(x, W_linmaps_id, w1)
  tm = _phase_b2((mq0, mq1, mq2, mq3), w_cat)
  qt, rt, invm = _phase_b3(xdom0, xdom1, cnt0, cnt1, w3, wv2,
                           W_linmaps_invariant)
  a_tab, y0, y1 = _phase_c1(tm, qt, rt, msd, mtd, z128)
  out0, out1 = _phase_c2(xw1, a_tab, srcp, iip, tgtp, z128)
  return _phase_d(base, out0, out1, invm, y0, y1,
                  domain_indicator.astype(I32))
